# baseline (speedup 1.0000x reference)
import functools
import math

import jax
import jax.numpy as jnp
from jax import lax
from jax.experimental import pallas as pl
from jax.experimental.pallas import tpu as pltpu


def _one_batch(x, wqkv, bqkv, wo, bo, causal01, *, n_heads, out_dtype):
    S, D = x.shape
    dh = D // n_heads
    xb = x.astype(jnp.bfloat16)
    qkv = (lax.dot_general(
        xb, wqkv, (((1,), (1,)), ((), ())),
        preferred_element_type=jnp.float32) + bqkv).astype(jnp.bfloat16)

    heads = []
    for h in range(n_heads):
        q = qkv[:, h * dh:(h + 1) * dh]
        k = qkv[:, D + h * dh:D + (h + 1) * dh]
        v = qkv[:, 2 * D + h * dh:2 * D + (h + 1) * dh]
        s = lax.dot_general(q, k, (((1,), (1,)), ((), ())),
                            preferred_element_type=jnp.float32)
        p = jnp.exp2(s) * causal01
        l = jnp.sum(p, axis=-1, keepdims=True)
        o = lax.dot_general(p.astype(jnp.bfloat16), v,
                            (((1,), (0,)), ((), ())),
                            preferred_element_type=jnp.float32)
        heads.append((o / l).astype(jnp.bfloat16))

    attn = jnp.concatenate(heads, axis=1)
    out = lax.dot_general(attn, wo, (((1,), (1,)), ((), ())),
                          preferred_element_type=jnp.float32) + bo
    return out.astype(out_dtype)


def _mhsa_kernel(x_ref, wqkv_ref, bqkv_ref, wo_ref, bo_ref, o_ref, *,
                 n_heads):
    nb = x_ref.shape[0]
    S = x_ref.shape[1]

    qi = lax.broadcasted_iota(jnp.int32, (S, S), 0)
    ki = lax.broadcasted_iota(jnp.int32, (S, S), 1)
    causal01 = (ki <= qi).astype(jnp.float32)

    for bb in range(nb):
        o_ref[bb] = _one_batch(
            x_ref[bb], wqkv_ref[...], bqkv_ref[...], wo_ref[...], bo_ref[...],
            causal01, n_heads=n_heads, out_dtype=o_ref.dtype)


def kernel(x, w_in, b_in, w_out, b_out):
    B, S, D = x.shape
    H = 12
    dh = D // H
    scale = math.log2(math.e) / math.sqrt(dh)
    nb = 2 if B % 2 == 0 else 1

    scale_col = jnp.concatenate(
        [jnp.full((D, 1), scale, jnp.float32),
         jnp.ones((2 * D, 1), jnp.float32)], axis=0)
    w_qkv = (w_in * scale_col).astype(jnp.bfloat16)
    b_qkv = (b_in * scale_col[:, 0]).reshape(1, 3 * D)
    wo = w_out.astype(jnp.bfloat16)
    bo = b_out.reshape(1, D)

    return pl.pallas_call(
        functools.partial(_mhsa_kernel, n_heads=H),
        out_shape=jax.ShapeDtypeStruct((B, S, D), x.dtype),
        grid=(B // nb,),
        in_specs=[
            pl.BlockSpec((nb, S, D), lambda b: (b, 0, 0)),
            pl.BlockSpec((3 * D, D), lambda b: (0, 0)),
            pl.BlockSpec((1, 3 * D), lambda b: (0, 0)),
            pl.BlockSpec((D, D), lambda b: (0, 0)),
            pl.BlockSpec((1, D), lambda b: (0, 0)),
        ],
        out_specs=pl.BlockSpec((nb, S, D), lambda b: (b, 0, 0)),
        compiler_params=pltpu.CompilerParams(
            dimension_semantics=("parallel",),
            vmem_limit_bytes=(56 << 20)),
    )(x, w_qkv, b_qkv, wo, bo)
